# Initial kernel scaffold; baseline (speedup 1.0000x reference)
import jax
import jax.numpy as jnp
from jax.experimental import pallas as pl


def kernel(x, edge_index, W0, b0, W1, b1, W2, b2):
    raise NotImplementedError("write your pallas kernel here")


# speedup vs baseline: 9.4456x; 9.4456x over previous
import functools

import jax
import jax.numpy as jnp
from jax import lax
from jax.experimental import pallas as pl
from jax.experimental.pallas import tpu as pltpu
from jax.experimental.pallas import tpu_sc as plsc

N = 10000
E = 320000
D = 128

NC = 2
NS = 16
CH = 128
NW = NC * NS

N_PAD = 10240
_EPT = -(-E // NW)
_ITERS = -(-_EPT // CH)
EPT = _ITERS * CH
E_PAD = EPT * NW
ROWS_PT = N_PAD // NS

_SC_PARAMS = dict(use_tc_tiling_on_sc=False)


@functools.cache
def _sc_degree_kernel():
    mesh = plsc.VectorSubcoreMesh(core_axis_name="c", subcore_axis_name="s")
    return pl.kernel(
        _sc_degree_body,
        out_type=jax.ShapeDtypeStruct((NC * N_PAD, D), jnp.float32),
        mesh=mesh,
        compiler_params=pltpu.CompilerParams(**_SC_PARAMS),
        scratch_types=[
            pltpu.VMEM((CH,), jnp.int32),
            pltpu.VMEM((CH, D), jnp.float32),
            pltpu.VMEM_SHARED((N_PAD, D), jnp.float32),
        ],
    )


def _sc_degree_body(dst_hbm, out_hbm, dst_v, rows_v, deg_sh):
    c = lax.axis_index("c")
    s = lax.axis_index("s")

    def _setrows(val):
        def _row(r, _):
            def _col(k, __):
                rows_v[r, pl.ds(k * 16, 16)] = jnp.full((16,), val,
                                                        jnp.float32)
                return __
            lax.fori_loop(0, D // 16, _col, None)
            return _
        lax.fori_loop(0, CH, _row, None)

    _setrows(0.0)

    def _zacc(j, _):
        pltpu.sync_copy(rows_v, deg_sh.at[pl.ds(s * ROWS_PT + j * CH, CH)])
        return _
    lax.fori_loop(0, ROWS_PT // CH, _zacc, None)
    _setrows(1.0)
    plsc.subcore_barrier()

    base = (c * NS + s) * EPT

    def _step(e, _):
        off = base + e * CH
        pltpu.sync_copy(dst_hbm.at[pl.ds(off, CH)], dst_v)
        pltpu.sync_copy(rows_v, deg_sh.at[dst_v], add=True)
        return _
    lax.fori_loop(0, _ITERS, _step, None)

    plsc.subcore_barrier()
    pltpu.sync_copy(deg_sh.at[pl.ds(s * ROWS_PT, ROWS_PT)],
                    out_hbm.at[pl.ds(c * N_PAD + s * ROWS_PT, ROWS_PT)])


@functools.cache
def _sc_scatter_kernel():
    mesh = plsc.VectorSubcoreMesh(core_axis_name="c", subcore_axis_name="s")
    return pl.kernel(
        _sc_scatter_body,
        out_type=jax.ShapeDtypeStruct((NC * N_PAD, D), jnp.float32),
        mesh=mesh,
        compiler_params=pltpu.CompilerParams(**_SC_PARAMS),
        scratch_types=[
            pltpu.VMEM((CH,), jnp.int32),
            pltpu.VMEM((CH,), jnp.int32),
            pltpu.VMEM((CH, D), jnp.float32),
            pltpu.VMEM_SHARED((N_PAD, D), jnp.float32),
            pltpu.SemaphoreType.DMA,
        ],
    )


def _sc_scatter_body(hs_hbm, src_hbm, dst_hbm, out_hbm, src_v, dst_v, rows_v,
                     acc_sh, sem):
    c = lax.axis_index("c")
    s = lax.axis_index("s")

    def _zrow(r, _):
        def _zcol(k, __):
            rows_v[r, pl.ds(k * 16, 16)] = jnp.zeros((16,), jnp.float32)
            return __
        lax.fori_loop(0, D // 16, _zcol, None)
        return _
    lax.fori_loop(0, CH, _zrow, None)

    def _zacc(j, _):
        pltpu.sync_copy(rows_v, acc_sh.at[pl.ds(s * ROWS_PT + j * CH, CH)])
        return _
    lax.fori_loop(0, ROWS_PT // CH, _zacc, None)
    plsc.subcore_barrier()

    base = (c * NS + s) * EPT

    def _step(e, _):
        off = base + e * CH
        pltpu.sync_copy(src_hbm.at[pl.ds(off, CH)], src_v)
        pltpu.sync_copy(dst_hbm.at[pl.ds(off, CH)], dst_v)
        pltpu.async_copy(hs_hbm.at[src_v], rows_v, sem).wait()
        pltpu.sync_copy(rows_v, acc_sh.at[dst_v], add=True)
        return _
    lax.fori_loop(0, _ITERS, _step, None)

    plsc.subcore_barrier()
    pltpu.sync_copy(acc_sh.at[pl.ds(s * ROWS_PT, ROWS_PT)],
                    out_hbm.at[pl.ds(c * N_PAD + s * ROWS_PT, ROWS_PT)])


BM = 1024


def _pre_body(x_ref, w_ref, degp_ref, hs_ref, dinv_ref):
    deg = (jnp.sum(degp_ref[0], axis=1)
           + jnp.sum(degp_ref[1], axis=1)) * (1.0 / D) + 1.0
    dinv = lax.rsqrt(deg)
    h = jnp.dot(x_ref[...], w_ref[...], preferred_element_type=jnp.float32)
    hs_ref[...] = h * dinv[:, None]
    dinv_ref[...] = dinv[:, None]


def _pre(x_pad, w0, degp):
    return pl.pallas_call(
        _pre_body,
        grid=(N_PAD // BM,),
        in_specs=[
            pl.BlockSpec((BM, D), lambda i: (i, 0)),
            pl.BlockSpec((D, D), lambda i: (0, 0)),
            pl.BlockSpec((NC, BM, D), lambda i: (0, i, 0)),
        ],
        out_specs=[
            pl.BlockSpec((BM, D), lambda i: (i, 0)),
            pl.BlockSpec((BM, 1), lambda i: (i, 0)),
        ],
        out_shape=[
            jax.ShapeDtypeStruct((N_PAD, D), jnp.float32),
            jax.ShapeDtypeStruct((N_PAD, 1), jnp.float32),
        ],
    )(x_pad, w0, degp)


def _mid_body(acc_ref, hs_ref, dinv_ref, b_ref, w_ref, out_ref):
    dinv = dinv_ref[...]
    o = (acc_ref[0] + acc_ref[1] + hs_ref[...]) * dinv + b_ref[...]
    x = jnp.maximum(o, 0.0)
    out_ref[...] = jnp.dot(x, w_ref[...],
                           preferred_element_type=jnp.float32) * dinv


def _mid(acc, hs, dinv, b, w_next):
    return pl.pallas_call(
        _mid_body,
        grid=(N_PAD // BM,),
        in_specs=[
            pl.BlockSpec((NC, BM, D), lambda i: (0, i, 0)),
            pl.BlockSpec((BM, D), lambda i: (i, 0)),
            pl.BlockSpec((BM, 1), lambda i: (i, 0)),
            pl.BlockSpec((1, D), lambda i: (0, 0)),
            pl.BlockSpec((D, D), lambda i: (0, 0)),
        ],
        out_specs=pl.BlockSpec((BM, D), lambda i: (i, 0)),
        out_shape=jax.ShapeDtypeStruct((N_PAD, D), jnp.float32),
    )(acc, hs, dinv, b, w_next)


def _post_body(acc_ref, hs_ref, dinv_ref, b_ref, out_ref):
    out_ref[...] = ((acc_ref[0] + acc_ref[1] + hs_ref[...]) * dinv_ref[...]
                    + b_ref[...])


def _post(acc, hs, dinv, b):
    return pl.pallas_call(
        _post_body,
        grid=(N_PAD // BM,),
        in_specs=[
            pl.BlockSpec((NC, BM, D), lambda i: (0, i, 0)),
            pl.BlockSpec((BM, D), lambda i: (i, 0)),
            pl.BlockSpec((BM, 1), lambda i: (i, 0)),
            pl.BlockSpec((1, D), lambda i: (0, 0)),
        ],
        out_specs=pl.BlockSpec((BM, D), lambda i: (i, 0)),
        out_shape=jax.ShapeDtypeStruct((N_PAD, D), jnp.float32),
    )(acc, hs, dinv, b)


def kernel(x, edge_index, W0, b0, W1, b1, W2, b2):
    x_pad = jnp.zeros((N_PAD, D), jnp.float32).at[:N].set(x)
    pad = jnp.full((E_PAD - E,), N, jnp.int32)
    src_p = jnp.concatenate([edge_index[0], pad])
    dst_p = jnp.concatenate([edge_index[1], pad])

    degp = _sc_degree_kernel()(dst_p).reshape(NC, N_PAD, D)
    hs, dinv = _pre(x_pad, W0, degp)
    acc = _sc_scatter_kernel()(hs, src_p, dst_p).reshape(NC, N_PAD, D)
    hs = _mid(acc, hs, dinv, b0.reshape(1, D), W1)
    acc = _sc_scatter_kernel()(hs, src_p, dst_p).reshape(NC, N_PAD, D)
    hs = _mid(acc, hs, dinv, b1.reshape(1, D), W2)
    acc = _sc_scatter_kernel()(hs, src_p, dst_p).reshape(NC, N_PAD, D)
    out = _post(acc, hs, dinv, b2.reshape(1, D))
    return out[:N]
